# baseline (speedup 1.0000x reference)
import functools

import jax
import jax.numpy as jnp
from jax import lax
from jax.experimental import pallas as pl
from jax.experimental.pallas import tpu as pltpu
from jax.experimental.pallas import tpu_sc as plsc

_EMB = 64
_ROWS = 17
_PTAB = _ROWS * _ROWS
_NC, _NS = 2, 16
_NW = _NC * _NS
_SUB = 128
_GPC = 2
_CHUNK = _SUB * _GPC
_L = 16


def _combine_body(layer_ref, dir_ref, out_ref):
    out_ref[...] = layer_ref[...] + dir_ref[...]


def _repack_body(h, m_ref, out_ref):
    x = m_ref[...]
    rb = x.shape[0]
    pad = jnp.zeros((rb, 2 * _SUB - h), jnp.int32)
    out_ref[...] = jnp.concatenate([x, pad], axis=1).reshape(2 * rb, _SUB)


_BPC = 4
_PPC = 4 * 100
_NGRP = _PPC // _L
_TAIL = _PPC - 3 * _SUB


def _sc_gather(b_rows, hp):
    rows_per_worker = b_rows // _NW
    n_chunks = rows_per_worker // _BPC
    n_iter = n_chunks // 2
    mesh = plsc.VectorSubcoreMesh(core_axis_name="c", subcore_axis_name="s")

    @functools.partial(
        pl.kernel,
        out_type=jax.ShapeDtypeStruct((b_rows * hp, 2 * _EMB), jnp.float32),
        mesh=mesh,
        scratch_types=[
            pltpu.VMEM((2 * _BPC, _SUB), jnp.int32),
            pltpu.VMEM((2 * _BPC, _SUB), jnp.int32),
            pltpu.VMEM((2, 4, _SUB), jnp.int32),
            pltpu.VMEM((2, 3 * _SUB + _TAIL, 2 * _EMB), jnp.float32),
            pltpu.SemaphoreType.DMA,
            pltpu.SemaphoreType.DMA,
        ],
    )
    def k(table_hbm, raw_hbm, out_hbm, raw_a, raw_b, idx_v, rows_v, sem_a, sem_b):
        wid = lax.axis_index("s") * _NC + lax.axis_index("c")
        row0 = wid * rows_per_worker
        tab0 = wid * _PTAB

        def scoped():
            lane = lax.iota(jnp.int32, _L)
            shift_idx = jnp.minimum(lane + 1, _L - 1)

            def take(v, idx):
                return lax.gather(
                    v,
                    idx[:, None],
                    dimension_numbers=lax.GatherDimensionNumbers(
                        offset_dims=(),
                        collapsed_slice_dims=(0,),
                        start_index_map=(0,),
                    ),
                    slice_sizes=(1,),
                    mode=lax.GatherScatterMode.PROMISE_IN_BOUNDS,
                )

            def fire(chunk, buf, sem):
                crow = row0 + chunk * _BPC
                raw = raw_a if buf == 0 else raw_b
                pltpu.sync_copy(raw_hbm.at[pl.ds(crow * 2, 2 * _BPC)], raw)

                def q_of(rho, g):
                    v = raw[rho, pl.ds(g * _L, _L)]
                    return v * _ROWS + take(v, shift_idx)

                for c in range(_NGRP):
                    segs = []
                    for lam in range(_L):
                        s = c * _L + lam
                        r, kq = divmod(s, hp)
                        rho = 2 * r + (1 if kq >= 64 else 0)
                        kk = kq - 64 if kq >= 64 else kq
                        g = kk // 8
                        if segs and segs[-1][1] == rho and segs[-1][2] == g:
                            continue
                        segs.append((lam, rho, g, kk))
                    acc = None
                    for lam0, rho, g, kk0 in segs:
                        m = (2 * (kk0 - lam0) + 2 * lane) & (_L - 1)
                        t = take(q_of(rho, g), m)
                        acc = t if acc is None else jnp.where(lane >= lam0, t, acc)
                    idx_v[buf, c // 8, pl.ds((c % 8) * _L, _L)] = acc + tab0

                for j in range(3):
                    pltpu.async_copy(
                        table_hbm.at[idx_v.at[buf].at[j]],
                        rows_v.at[buf].at[pl.ds(j * _SUB, _SUB)],
                        sem,
                    )
                pltpu.async_copy(
                    table_hbm.at[idx_v.at[buf].at[3].at[pl.ds(0, _TAIL)]],
                    rows_v.at[buf].at[pl.ds(3 * _SUB, _TAIL)],
                    sem,
                )

            def drain_and_out(chunk, buf, sem):
                for j in range(3):
                    pltpu.make_async_copy(
                        table_hbm.at[idx_v.at[buf].at[j]],
                        rows_v.at[buf].at[pl.ds(j * _SUB, _SUB)],
                        sem,
                    ).wait()
                pltpu.make_async_copy(
                    table_hbm.at[idx_v.at[buf].at[3].at[pl.ds(0, _TAIL)]],
                    rows_v.at[buf].at[pl.ds(3 * _SUB, _TAIL)],
                    sem,
                ).wait()
                crow = row0 + chunk * _BPC
                pltpu.sync_copy(
                    rows_v.at[buf].at[pl.ds(0, _PPC)],
                    out_hbm.at[pl.ds(crow * hp, _PPC)],
                )

            fire(0, 0, sem_a)

            def body(i, carry):
                g = 2 * i
                fire(g + 1, 1, sem_b)
                drain_and_out(g, 0, sem_a)

                @pl.when(i < n_iter - 1)
                def _prefetch():
                    fire(g + 2, 0, sem_a)

                drain_and_out(g + 1, 1, sem_b)
                return carry

            lax.fori_loop(0, n_iter, body, 0)

        scoped()

    return k


def kernel(metal_layer, layer_table, direction_table):
    b, h = metal_layer.shape
    hp = h // 2

    layer_pad = jnp.pad(layer_table, ((0, 18 - _ROWS), (0, 0)))
    dir_tiled = jnp.tile(direction_table, (9, 1))
    combined = pl.pallas_call(
        _combine_body,
        out_shape=jax.ShapeDtypeStruct((18, _EMB), jnp.float32),
    )(layer_pad, dir_tiled)[:_ROWS]

    left = jnp.repeat(combined, _ROWS, axis=0)
    right = jnp.tile(combined, (_ROWS, 1))
    pair_table = jnp.tile(jnp.concatenate([left, right], axis=1), (_NW, 1))

    rb = 256
    repacked = pl.pallas_call(
        functools.partial(_repack_body, h),
        grid=(b // rb,),
        in_specs=[pl.BlockSpec((rb, h), lambda i: (i, 0))],
        out_specs=pl.BlockSpec((2 * rb, _SUB), lambda i: (i, 0)),
        out_shape=jax.ShapeDtypeStruct((2 * b, _SUB), jnp.int32),
    )(metal_layer)

    out = _sc_gather(b, hp)(pair_table, repacked)
    return out.reshape(b, h, _EMB)


# speedup vs baseline: 1.0006x; 1.0006x over previous
import functools

import jax
import jax.numpy as jnp
from jax import lax
from jax.experimental import pallas as pl
from jax.experimental.pallas import tpu as pltpu
from jax.experimental.pallas import tpu_sc as plsc

_EMB = 64
_ROWS = 17
_PTAB = _ROWS * _ROWS
_NC, _NS = 2, 16
_NW = _NC * _NS
_SUB = 128
_GPC = 2
_CHUNK = _SUB * _GPC
_L = 16


def _combine_body(layer_ref, dir_ref, out_ref):
    out_ref[...] = layer_ref[...] + dir_ref[...]


def _repack_body(h, m_ref, out_ref):
    x = m_ref[...]
    rb = x.shape[0]
    pad = jnp.zeros((rb, 2 * _SUB - h), jnp.int32)
    out_ref[...] = jnp.concatenate([x, pad], axis=1).reshape(2 * rb, _SUB)


_BPC = 4
_PPC = 4 * 100
_NGRP = _PPC // _L
_TAIL = _PPC - 3 * _SUB


def _sc_gather(b_rows, hp):
    rows_per_worker = b_rows // _NW
    n_chunks = rows_per_worker // _BPC
    n_iter = n_chunks // 2
    mesh = plsc.VectorSubcoreMesh(core_axis_name="c", subcore_axis_name="s")

    @functools.partial(
        pl.kernel,
        out_type=jax.ShapeDtypeStruct((b_rows * hp, 2 * _EMB), jnp.float32),
        mesh=mesh,
        compiler_params=pltpu.CompilerParams(use_tc_tiling_on_sc=True),
        scratch_types=[
            pltpu.VMEM((2 * _BPC, _SUB), jnp.int32),
            pltpu.VMEM((2 * _BPC, _SUB), jnp.int32),
            pltpu.VMEM((2, 4, _SUB), jnp.int32),
            pltpu.VMEM((2, 3 * _SUB + _TAIL, 2 * _EMB), jnp.float32),
            pltpu.SemaphoreType.DMA,
            pltpu.SemaphoreType.DMA,
        ],
    )
    def k(table_hbm, raw_hbm, out_hbm, raw_a, raw_b, idx_v, rows_v, sem_a, sem_b):
        wid = lax.axis_index("s") * _NC + lax.axis_index("c")
        row0 = wid * rows_per_worker
        tab0 = wid * _PTAB

        def scoped():
            lane = lax.iota(jnp.int32, _L)
            shift_idx = jnp.minimum(lane + 1, _L - 1)

            def take(v, idx):
                return lax.gather(
                    v,
                    idx[:, None],
                    dimension_numbers=lax.GatherDimensionNumbers(
                        offset_dims=(),
                        collapsed_slice_dims=(0,),
                        start_index_map=(0,),
                    ),
                    slice_sizes=(1,),
                    mode=lax.GatherScatterMode.PROMISE_IN_BOUNDS,
                )

            def fire(chunk, buf, sem):
                crow = row0 + chunk * _BPC
                raw = raw_a if buf == 0 else raw_b
                pltpu.sync_copy(raw_hbm.at[pl.ds(crow * 2, 2 * _BPC)], raw)

                def q_of(rho, g):
                    v = raw[rho, pl.ds(g * _L, _L)]
                    return v * _ROWS + take(v, shift_idx)

                for c in range(_NGRP):
                    segs = []
                    for lam in range(_L):
                        s = c * _L + lam
                        r, kq = divmod(s, hp)
                        rho = 2 * r + (1 if kq >= 64 else 0)
                        kk = kq - 64 if kq >= 64 else kq
                        g = kk // 8
                        if segs and segs[-1][1] == rho and segs[-1][2] == g:
                            continue
                        segs.append((lam, rho, g, kk))
                    acc = None
                    for lam0, rho, g, kk0 in segs:
                        m = (2 * (kk0 - lam0) + 2 * lane) & (_L - 1)
                        t = take(q_of(rho, g), m)
                        acc = t if acc is None else jnp.where(lane >= lam0, t, acc)
                    idx_v[buf, c // 8, pl.ds((c % 8) * _L, _L)] = acc + tab0

                for j in range(3):
                    pltpu.async_copy(
                        table_hbm.at[idx_v.at[buf].at[j]],
                        rows_v.at[buf].at[pl.ds(j * _SUB, _SUB)],
                        sem,
                    )
                pltpu.async_copy(
                    table_hbm.at[idx_v.at[buf].at[3].at[pl.ds(0, _TAIL)]],
                    rows_v.at[buf].at[pl.ds(3 * _SUB, _TAIL)],
                    sem,
                )

            def drain_and_out(chunk, buf, sem):
                for j in range(3):
                    pltpu.make_async_copy(
                        table_hbm.at[idx_v.at[buf].at[j]],
                        rows_v.at[buf].at[pl.ds(j * _SUB, _SUB)],
                        sem,
                    ).wait()
                pltpu.make_async_copy(
                    table_hbm.at[idx_v.at[buf].at[3].at[pl.ds(0, _TAIL)]],
                    rows_v.at[buf].at[pl.ds(3 * _SUB, _TAIL)],
                    sem,
                ).wait()
                crow = row0 + chunk * _BPC
                pltpu.sync_copy(
                    rows_v.at[buf].at[pl.ds(0, _PPC)],
                    out_hbm.at[pl.ds(crow * hp, _PPC)],
                )

            fire(0, 0, sem_a)

            def body(i, carry):
                g = 2 * i
                fire(g + 1, 1, sem_b)
                drain_and_out(g, 0, sem_a)

                @pl.when(i < n_iter - 1)
                def _prefetch():
                    fire(g + 2, 0, sem_a)

                drain_and_out(g + 1, 1, sem_b)
                return carry

            lax.fori_loop(0, n_iter, body, 0)

        scoped()

    return k


def kernel(metal_layer, layer_table, direction_table):
    b, h = metal_layer.shape
    hp = h // 2

    layer_pad = jnp.pad(layer_table, ((0, 18 - _ROWS), (0, 0)))
    dir_tiled = jnp.tile(direction_table, (9, 1))
    combined = pl.pallas_call(
        _combine_body,
        out_shape=jax.ShapeDtypeStruct((18, _EMB), jnp.float32),
    )(layer_pad, dir_tiled)[:_ROWS]

    left = jnp.repeat(combined, _ROWS, axis=0)
    right = jnp.tile(combined, (_ROWS, 1))
    pair_table = jnp.tile(jnp.concatenate([left, right], axis=1), (_NW, 1))

    rb = 256
    repacked = pl.pallas_call(
        functools.partial(_repack_body, h),
        grid=(b // rb,),
        in_specs=[pl.BlockSpec((rb, h), lambda i: (i, 0))],
        out_specs=pl.BlockSpec((2 * rb, _SUB), lambda i: (i, 0)),
        out_shape=jax.ShapeDtypeStruct((2 * b, _SUB), jnp.int32),
    )(metal_layer)

    out = _sc_gather(b, hp)(pair_table, repacked)
    return out.reshape(b, h, _EMB)
